# baseline (speedup 1.0000x reference)
import functools

import jax
import jax.numpy as jnp
from jax import lax
from jax.experimental import pallas as pl
from jax.experimental.pallas import tpu as pltpu
from jax.experimental.pallas import tpu_sc as plsc

L = 16
NC, NS = 2, 16
A_IN = 60
A_OUT = 60
K = 6
NB = 4
M_WORDS = A_IN * A_OUT


def _build_mt_sc():
    mesh = plsc.VectorSubcoreMesh(core_axis_name="c", subcore_axis_name="s",
                                  num_cores=NC, num_subcores=NS)

    @functools.partial(
        pl.kernel,
        out_type=jax.ShapeDtypeStruct((M_WORDS,), jnp.float32),
        mesh=mesh,
        scratch_types=[
            pltpu.VMEM((M_WORDS,), jnp.float32),
            pltpu.VMEM((K * NB * L,), jnp.int32),
            pltpu.VMEM((K * NB * L,), jnp.float32),
        ],
        compiler_params=pltpu.CompilerParams(needs_layout_passes=False),
    )
    def k(idxt_hbm, wt_hbm, mt_hbm, m_v, idx_v, w_v):
        wid = lax.axis_index("s") * NC + lax.axis_index("c")

        @pl.when(wid == 0)
        def _():
            pltpu.sync_copy(idxt_hbm, idx_v)
            pltpu.sync_copy(wt_hbm, w_v)
            zero = jnp.zeros((L,), jnp.float32)

            def zero_body(i, _):
                m_v[pl.ds(i * L, L)] = zero
                return ()

            lax.fori_loop(0, M_WORDS // L, zero_body, (), unroll=4)

            lane = lax.iota(jnp.int32, L)
            for b in range(NB):
                ao = lane + b * L
                mask = ao < A_OUT
                for j in range(K):
                    a_j = idx_v[pl.ds((j * NB + b) * L, L)]
                    w_j = w_v[pl.ds((j * NB + b) * L, L)]
                    plsc.store_scatter(m_v, [a_j * A_OUT + ao], w_j,
                                       mask=mask)
            pltpu.sync_copy(m_v, mt_hbm)

    return k


def _matmul_2d(rows, blk):
    def body(x_ref, m_ref, o_ref):
        o_ref[...] = jnp.dot(x_ref[...], m_ref[...],
                             preferred_element_type=jnp.float32)

    return pl.pallas_call(
        body,
        grid=(rows // blk,),
        in_specs=[
            pl.BlockSpec((blk, A_IN), lambda i: (i, 0)),
            pl.BlockSpec((A_IN, A_OUT), lambda i: (0, 0)),
        ],
        out_specs=pl.BlockSpec((blk, A_OUT), lambda i: (i, 0)),
        out_shape=jax.ShapeDtypeStruct((rows, A_OUT), jnp.float32),
    )


@jax.jit
def kernel(xyz, feats, idx, w, anchor_out):
    B, C, N, A = feats.shape
    idx_t = jnp.zeros((K, NB * L), jnp.int32).at[:, :A_OUT].set(
        idx.astype(jnp.int32).T)
    w_t = jnp.zeros((K, NB * L), jnp.float32).at[:, :A_OUT].set(w.T)
    mt = _build_mt_sc()(idx_t.reshape(-1), w_t.reshape(-1))
    rows = B * C * N
    out2 = _matmul_2d(rows, 4096)(feats.reshape(rows, A), mt.reshape(A_IN, A_OUT))
    return (xyz, out2.reshape(B, C, N, A_OUT), anchor_out)


# speedup vs baseline: 1.0655x; 1.0655x over previous
import functools

import jax
import jax.numpy as jnp
from jax import lax
from jax.experimental import pallas as pl
from jax.experimental.pallas import tpu as pltpu
from jax.experimental.pallas import tpu_sc as plsc

L = 16
NC, NS = 2, 16
A_IN = 60
A_OUT = 60
K = 6
NB = 4
M_WORDS = A_IN * A_OUT


def _build_mt_sc():
    mesh = plsc.VectorSubcoreMesh(core_axis_name="c", subcore_axis_name="s",
                                  num_cores=NC, num_subcores=NS)

    @functools.partial(
        pl.kernel,
        out_type=jax.ShapeDtypeStruct((M_WORDS,), jnp.float32),
        mesh=mesh,
        scratch_types=[
            pltpu.VMEM((M_WORDS,), jnp.float32),
            pltpu.VMEM((K * NB * L,), jnp.int32),
            pltpu.VMEM((K * NB * L,), jnp.float32),
        ],
        compiler_params=pltpu.CompilerParams(needs_layout_passes=False),
    )
    def k(idxt_hbm, wt_hbm, mt_hbm, m_v, idx_v, w_v):
        wid = lax.axis_index("s") * NC + lax.axis_index("c")

        @pl.when(wid == 0)
        def _():
            pltpu.sync_copy(idxt_hbm, idx_v)
            pltpu.sync_copy(wt_hbm, w_v)
            zero = jnp.zeros((L,), jnp.float32)

            def zero_body(i, _):
                m_v[pl.ds(i * L, L)] = zero
                return ()

            lax.fori_loop(0, M_WORDS // L, zero_body, (), unroll=4)

            lane = lax.iota(jnp.int32, L)
            for b in range(NB):
                ao = lane + b * L
                mask = ao < A_OUT
                for j in range(K):
                    a_j = idx_v[pl.ds((j * NB + b) * L, L)]
                    w_j = w_v[pl.ds((j * NB + b) * L, L)]
                    plsc.store_scatter(m_v, [a_j * A_OUT + ao], w_j,
                                       mask=mask)
            pltpu.sync_copy(m_v, mt_hbm)

    return k


def _matmul_2d(rows, blk):
    def body(x_ref, m_ref, o_ref):
        o_ref[...] = jnp.dot(x_ref[...], m_ref[...],
                             preferred_element_type=jnp.float32)

    return pl.pallas_call(
        body,
        grid=(rows // blk,),
        in_specs=[
            pl.BlockSpec((blk, A_IN), lambda i: (i, 0)),
            pl.BlockSpec((A_IN, A_OUT), lambda i: (0, 0)),
        ],
        out_specs=pl.BlockSpec((blk, A_OUT), lambda i: (i, 0)),
        out_shape=jax.ShapeDtypeStruct((rows, A_OUT), jnp.float32),
    )


@jax.jit
def kernel(xyz, feats, idx, w, anchor_out):
    B, C, N, A = feats.shape
    idx_t = jnp.zeros((K, NB * L), jnp.int32).at[:, :A_OUT].set(
        idx.astype(jnp.int32).T)
    w_t = jnp.zeros((K, NB * L), jnp.float32).at[:, :A_OUT].set(w.T)
    mt = _build_mt_sc()(idx_t.reshape(-1), w_t.reshape(-1))
    rows = B * C * N
    out2 = _matmul_2d(rows, 16384)(feats.reshape(rows, A), mt.reshape(A_IN, A_OUT))
    return (xyz, out2.reshape(B, C, N, A_OUT), anchor_out)
